# baseline (device time: 139335 ns/iter reference)
import jax
import jax.numpy as jnp
from jax import lax
from jax.experimental import pallas as pl
from jax.experimental.pallas import tpu as pltpu

M = 2048
D = 2048
K = 8192
M_HALF = M // 2

P = 2
RP = M_HALF // P
NC = 4
DC = D // NC
G = P * NC
KH = K // 2
SUB = 1
DS = DC // SUB
NSLOT = 4


def _body(dy_hbm, w_hbm, out_ref,
          dy_vmem, w_buf, partial, recv_y,
          load_sems, y_send, y_recv, x_send, x_recv):
    my_x = lax.axis_index("x")
    my_y = lax.axis_index("y")
    y_peer = (my_x, 1 - my_y)
    x_peer = (1 - my_x, my_y)
    row0 = my_x * M_HALF

    dy_copies = [
        pltpu.make_async_copy(
            dy_hbm.at[pl.ds(row0 + p * RP, RP)], dy_vmem, load_sems.at[0])
        for p in range(P)
    ]
    dy_copies[0].start()

    w_copies = [None] * (2 * G)

    def start_w_load(s):
        c, t = divmod(s, 2)
        cp = pltpu.make_async_copy(
            w_hbm.at[pl.ds((c % NC) * DC, DC), pl.ds(t * KH, KH)],
            w_buf.at[s % 2], load_sems.at[1 + (s % 2)])
        cp.start()
        w_copies[s] = cp

    start_w_load(0)
    start_w_load(1)

    barrier = pltpu.get_barrier_semaphore()
    for peer in (y_peer, x_peer):
        pl.semaphore_signal(
            barrier, inc=1, device_id=peer,
            device_id_type=pl.DeviceIdType.MESH,
        )
    pl.semaphore_wait(barrier, 2)

    dy_copies[0].wait()

    rdmas_y = [None] * (SUB * G)
    rdmas_x = [None] * (SUB * G)

    def sub_block(u, half_row0):
        c, i = divmod(u, SUB)
        p, j = divmod(c, NC)
        return (pl.ds(half_row0 + p * RP, RP), pl.ds(j * DC + i * DS, DS))

    def process(u):
        c, i = divmod(u, SUB)
        rdmas_y[u].wait_recv()
        blk = sub_block(u, row0)
        out_ref[blk] = (
            partial[c % NSLOT, :, pl.ds(i * DS, DS)]
            + recv_y[c, :, pl.ds(i * DS, DS)]
        )
        rx = pltpu.make_async_remote_copy(
            src_ref=out_ref.at[blk],
            dst_ref=out_ref.at[blk],
            send_sem=x_send.at[u],
            recv_sem=x_recv.at[u],
            device_id=x_peer,
            device_id_type=pl.DeviceIdType.MESH,
        )
        rx.start()
        rdmas_x[u] = rx

    for c in range(G):
        slot = c % NSLOT
        if c >= NSLOT:
            for i in range(SUB):
                rdmas_y[SUB * (c - NSLOT) + i].wait_send()
        for t in range(2):
            s = 2 * c + t
            w_copies[s].wait()
            if c == NC and t == 0:
                dy_copies[1].wait()
            d = lax.dot_general(
                dy_vmem[:, pl.ds(t * KH, KH)], w_buf[s % 2],
                dimension_numbers=(((1,), (1,)), ((), ())),
                preferred_element_type=jnp.float32,
            )
            partial[slot] = d if t == 0 else partial[slot] + d
            if s + 2 < 2 * G:
                start_w_load(s + 2)
            if s == 2 * NC - 1:
                dy_copies[1].start()
        for i in range(SUB):
            u = SUB * c + i
            ry = pltpu.make_async_remote_copy(
                src_ref=partial.at[slot, slice(None), pl.ds(i * DS, DS)],
                dst_ref=recv_y.at[c, slice(None), pl.ds(i * DS, DS)],
                send_sem=y_send.at[u],
                recv_sem=y_recv.at[u],
                device_id=y_peer,
                device_id_type=pl.DeviceIdType.MESH,
            )
            ry.start()
            rdmas_y[u] = ry
        if c >= 1:
            for i in range(SUB):
                process(SUB * (c - 1) + i)
    for i in range(SUB):
        process(SUB * (G - 1) + i)

    for u in range(SUB * G):
        rdmas_x[u].wait_recv()
    for u in range(SUB * (G - NSLOT), SUB * G):
        rdmas_y[u].wait_send()
    for u in range(SUB * G):
        rdmas_x[u].wait_send()


def kernel(dy, W):
    return pl.pallas_call(
        _body,
        out_shape=jax.ShapeDtypeStruct((M, D), jnp.float32),
        in_specs=[
            pl.BlockSpec(memory_space=pl.ANY),
            pl.BlockSpec(memory_space=pl.ANY),
        ],
        out_specs=pl.BlockSpec(memory_space=pltpu.VMEM),
        scratch_shapes=[
            pltpu.VMEM((RP, K), jnp.float32),
            pltpu.VMEM((2, DC, KH), jnp.float32),
            pltpu.VMEM((NSLOT, RP, DC), jnp.float32),
            pltpu.VMEM((G, RP, DC), jnp.float32),
            pltpu.SemaphoreType.DMA((3,)),
            pltpu.SemaphoreType.DMA((SUB * G,)),
            pltpu.SemaphoreType.DMA((SUB * G,)),
            pltpu.SemaphoreType.DMA((SUB * G,)),
            pltpu.SemaphoreType.DMA((SUB * G,)),
        ],
        compiler_params=pltpu.CompilerParams(
            collective_id=0,
            vmem_limit_bytes=62 * 1024 * 1024,
        ),
    )(dy, W)


# device time: 135681 ns/iter; 1.0269x vs baseline; 1.0269x over previous
import jax
import jax.numpy as jnp
from jax import lax
from jax.experimental import pallas as pl
from jax.experimental.pallas import tpu as pltpu

M = 2048
D = 2048
K = 8192
M_HALF = M // 2

P = 2
RP = M_HALF // P
NC = 4
DC = D // NC
G = P * NC
KH = K // 2
SUB = 2
DS = DC // SUB
NSLOT = 4


def _body(dy_hbm, w_hbm, out_ref,
          dy_vmem, w_buf, partial, recv_y,
          load_sems, y_send, y_recv, x_send, x_recv):
    my_x = lax.axis_index("x")
    my_y = lax.axis_index("y")
    y_peer = (my_x, 1 - my_y)
    x_peer = (1 - my_x, my_y)
    row0 = my_x * M_HALF

    dy_copies = [
        pltpu.make_async_copy(
            dy_hbm.at[pl.ds(row0 + p * RP, RP)], dy_vmem, load_sems.at[0])
        for p in range(P)
    ]
    dy_copies[0].start()

    w_copies = [None] * (2 * G)

    def start_w_load(s):
        c, t = divmod(s, 2)
        cp = pltpu.make_async_copy(
            w_hbm.at[pl.ds((c % NC) * DC, DC), pl.ds(t * KH, KH)],
            w_buf.at[s % 2], load_sems.at[1 + (s % 2)])
        cp.start()
        w_copies[s] = cp

    start_w_load(0)
    start_w_load(1)

    barrier = pltpu.get_barrier_semaphore()
    for peer in (y_peer, x_peer):
        pl.semaphore_signal(
            barrier, inc=1, device_id=peer,
            device_id_type=pl.DeviceIdType.MESH,
        )
    pl.semaphore_wait(barrier, 2)

    dy_copies[0].wait()

    rdmas_y = [None] * (SUB * G)
    rdmas_x = [None] * (SUB * G)

    def sub_block(u, half_row0):
        c, i = divmod(u, SUB)
        p, j = divmod(c, NC)
        return (pl.ds(half_row0 + p * RP, RP), pl.ds(j * DC + i * DS, DS))

    def process(u):
        c, i = divmod(u, SUB)
        rdmas_y[u].wait_recv()
        blk = sub_block(u, row0)
        out_ref[blk] = (
            partial[c % NSLOT, :, pl.ds(i * DS, DS)]
            + recv_y[c, :, pl.ds(i * DS, DS)]
        )
        rx = pltpu.make_async_remote_copy(
            src_ref=out_ref.at[blk],
            dst_ref=out_ref.at[blk],
            send_sem=x_send.at[u],
            recv_sem=x_recv.at[u],
            device_id=x_peer,
            device_id_type=pl.DeviceIdType.MESH,
        )
        rx.start()
        rdmas_x[u] = rx

    for c in range(G):
        slot = c % NSLOT
        if c >= NSLOT:
            for i in range(SUB):
                rdmas_y[SUB * (c - NSLOT) + i].wait_send()
        for t in range(2):
            s = 2 * c + t
            w_copies[s].wait()
            if c == NC and t == 0:
                dy_copies[1].wait()
            d = lax.dot_general(
                dy_vmem[:, pl.ds(t * KH, KH)], w_buf[s % 2],
                dimension_numbers=(((1,), (1,)), ((), ())),
                preferred_element_type=jnp.float32,
            )
            partial[slot] = d if t == 0 else partial[slot] + d
            if s + 2 < 2 * G:
                start_w_load(s + 2)
            if s == 2 * NC - 1:
                dy_copies[1].start()
        for i in range(SUB):
            u = SUB * c + i
            ry = pltpu.make_async_remote_copy(
                src_ref=partial.at[slot, slice(None), pl.ds(i * DS, DS)],
                dst_ref=recv_y.at[c, slice(None), pl.ds(i * DS, DS)],
                send_sem=y_send.at[u],
                recv_sem=y_recv.at[u],
                device_id=y_peer,
                device_id_type=pl.DeviceIdType.MESH,
            )
            ry.start()
            rdmas_y[u] = ry
        if c >= 1:
            for i in range(SUB):
                process(SUB * (c - 1) + i)
    for i in range(SUB):
        process(SUB * (G - 1) + i)

    for u in range(SUB * G):
        rdmas_x[u].wait_recv()
    for u in range(SUB * (G - NSLOT), SUB * G):
        rdmas_y[u].wait_send()
    for u in range(SUB * G):
        rdmas_x[u].wait_send()


def kernel(dy, W):
    return pl.pallas_call(
        _body,
        out_shape=jax.ShapeDtypeStruct((M, D), jnp.float32),
        in_specs=[
            pl.BlockSpec(memory_space=pl.ANY),
            pl.BlockSpec(memory_space=pl.ANY),
        ],
        out_specs=pl.BlockSpec(memory_space=pltpu.VMEM),
        scratch_shapes=[
            pltpu.VMEM((RP, K), jnp.float32),
            pltpu.VMEM((2, DC, KH), jnp.float32),
            pltpu.VMEM((NSLOT, RP, DC), jnp.float32),
            pltpu.VMEM((G, RP, DC), jnp.float32),
            pltpu.SemaphoreType.DMA((3,)),
            pltpu.SemaphoreType.DMA((SUB * G,)),
            pltpu.SemaphoreType.DMA((SUB * G,)),
            pltpu.SemaphoreType.DMA((SUB * G,)),
            pltpu.SemaphoreType.DMA((SUB * G,)),
        ],
        compiler_params=pltpu.CompilerParams(
            collective_id=0,
            vmem_limit_bytes=62 * 1024 * 1024,
        ),
    )(dy, W)
